# Initial kernel scaffold; baseline (speedup 1.0000x reference)
import jax
import jax.numpy as jnp
from jax.experimental import pallas as pl


def kernel(X, emb):
    raise NotImplementedError("write your pallas kernel here")


# speedup vs baseline: 1.8375x; 1.8375x over previous
import functools

import jax
import jax.numpy as jnp
from jax import lax
from jax.experimental import pallas as pl
from jax.experimental.pallas import tpu as pltpu
from jax.experimental.pallas import tpu_sc as plsc

_NC = 2
_NS = 16
_NW = _NC * _NS

_CHUNK = 128
_NBUF = 2


def _gather_body(idx_hbm, table_hbm, out_hbm, idx_v, rows_v, sems,
                 *, n_chunks):
  wid = lax.axis_index("s") * _NC + lax.axis_index("c")
  pltpu.sync_copy(idx_hbm.at[wid], idx_v)
  row_base = wid * n_chunks * _CHUNK

  def fire(j, b):
    pltpu.async_copy(table_hbm.at[idx_v.at[j]], rows_v.at[b], sems.at[b])

  def drain(j, b):
    pltpu.make_async_copy(table_hbm.at[idx_v.at[j]], rows_v.at[b],
                          sems.at[b]).wait()
    pltpu.sync_copy(rows_v.at[b],
                    out_hbm.at[pl.ds(row_base + j * _CHUNK, _CHUNK)])

  for b in range(_NBUF):
    fire(b, b)

  n_steady = (n_chunks - _NBUF) // _NBUF

  def loop_body(i, _):
    j0 = i * _NBUF
    for b in range(_NBUF):
      drain(j0 + b, b)
      fire(j0 + b + _NBUF, b)
    return ()

  lax.fori_loop(0, n_steady, loop_body, ())

  for t in range(_NBUF):
    drain(n_chunks - _NBUF + t, t)


def kernel(X, emb):
  B, S = X.shape
  V, D = emb.shape
  n_rows = B * S
  assert n_rows % (_NW * _CHUNK * _NBUF) == 0
  n_chunks = n_rows // (_NW * _CHUNK)

  idx = X.reshape(_NW, n_chunks, _CHUNK).astype(jnp.int32)

  mesh = plsc.VectorSubcoreMesh(core_axis_name="c", subcore_axis_name="s")
  body = functools.partial(_gather_body, n_chunks=n_chunks)
  out = pl.kernel(
      body,
      out_type=jax.ShapeDtypeStruct((n_rows, D), jnp.float32),
      mesh=mesh,
      compiler_params=pltpu.CompilerParams(use_tc_tiling_on_sc=False),
      scratch_types=[
          pltpu.VMEM((n_chunks, _CHUNK), jnp.int32),
          pltpu.VMEM((_NBUF, _CHUNK, D), jnp.float32),
          pltpu.SemaphoreType.DMA((_NBUF,)),
      ],
  )(idx, emb)
  return out.reshape(B, S, D)
